# speedup vs baseline: 255.1691x; 255.1691x over previous
import functools

import jax
import jax.numpy as jnp
from jax import lax
from jax.experimental import pallas as pl
from jax.experimental.pallas import tpu as pltpu
from jax.experimental.pallas import tpu_sc as plsc

N = 1024
D = 64
E = 200000

_NW = 32
_LANES = 16
_E_PAD = 200704
_CHUNK = _E_PAD // _NW


def _tc_body(z_ref, w0_ref, b0_ref, w1_ref, b1_ref, bias_ref, s_ref):
    z = z_ref[...]
    zbar = jnp.mean(z, axis=0, keepdims=True)
    c0 = jax.nn.relu(
        jnp.dot(zbar, w0_ref[...], preferred_element_type=jnp.float32)
        + b0_ref[...])
    c1 = (jnp.dot(zbar + c0, w1_ref[...], preferred_element_type=jnp.float32)
          + b1_ref[...])
    h = z + c0 + c1
    g = lax.dot_general(h, h, (((1,), (1,)), ((), ())),
                        preferred_element_type=jnp.float32)
    s_ref[...] = jax.nn.sigmoid(g + bias_ref[0, 0])


def _sc_gather(s_hbm, src_hbm, dst_hbm, out_hbm, src_v, dst_v, idx_v, val_v,
               sem):
    nc = 2
    wid = lax.axis_index("s") * nc + lax.axis_index("c")
    base = wid * _CHUNK
    pltpu.sync_copy(src_hbm.at[pl.ds(base, _CHUNK)], src_v)
    pltpu.sync_copy(dst_hbm.at[pl.ds(base, _CHUNK)], dst_v)

    def body(i, carry):
        sl = pl.ds(i * _LANES, _LANES)
        idx_v[sl] = src_v[sl] * N + dst_v[sl]
        return carry

    lax.fori_loop(0, _CHUNK // _LANES, body, 0)
    pltpu.async_copy(s_hbm.at[idx_v], val_v, sem).wait()
    pltpu.sync_copy(val_v, out_hbm.at[pl.ds(base, _CHUNK)])


def kernel(z, edge_index, W0, b0, W1, b1, bias):
    s = pl.pallas_call(
        _tc_body,
        out_shape=jax.ShapeDtypeStruct((N, N), jnp.float32),
    )(z, W0, b0.reshape(1, D), W1, b1.reshape(1, D), bias.reshape(1, 1))

    s_flat = s.reshape(N * N)
    src = jnp.pad(edge_index[0], (0, _E_PAD - E))
    dst = jnp.pad(edge_index[1], (0, _E_PAD - E))

    mesh = plsc.VectorSubcoreMesh(core_axis_name="c", subcore_axis_name="s")
    gather = functools.partial(
        pl.kernel,
        mesh=mesh,
        out_type=jax.ShapeDtypeStruct((_E_PAD,), jnp.float32),
        scratch_types=[
            pltpu.VMEM((_CHUNK,), jnp.int32),
            pltpu.VMEM((_CHUNK,), jnp.int32),
            pltpu.VMEM((_CHUNK,), jnp.int32),
            pltpu.VMEM((_CHUNK,), jnp.float32),
            pltpu.SemaphoreType.DMA,
        ],
    )(_sc_gather)

    out = gather(s_flat, src, dst)
    return out[:E]


# speedup vs baseline: 300.1369x; 1.1762x over previous
import functools

import jax
import jax.numpy as jnp
from jax import lax
from jax.experimental import pallas as pl
from jax.experimental.pallas import tpu as pltpu
from jax.experimental.pallas import tpu_sc as plsc

N = 1024
D = 64
E = 200000

_NW = 32
_LANES = 16
_E_PAD = 200704
_CHUNK = _E_PAD // _NW


def _tc_body(z_ref, ei_ref, w0_ref, b0_ref, w1_ref, b1_ref, bias_ref, s_ref,
             idx_ref):
    z = z_ref[...]
    zbar = jnp.mean(z, axis=0, keepdims=True)
    c0 = jax.nn.relu(
        jnp.dot(zbar, w0_ref[...], preferred_element_type=jnp.float32)
        + b0_ref[...])
    c1 = (jnp.dot(zbar + c0, w1_ref[...], preferred_element_type=jnp.float32)
          + b1_ref[...])
    h = z + c0 + c1
    g = lax.dot_general(h, h, (((1,), (1,)), ((), ())),
                        preferred_element_type=jnp.float32)
    s_ref[...] = jax.nn.sigmoid(g + bias_ref[0, 0])
    idx_ref[...] = ei_ref[0] * N + ei_ref[1]


def _sc_gather(s_hbm, idx_hbm, out_hbm, idx_v, val_v, sem):
    nc = 2
    wid = lax.axis_index("s") * nc + lax.axis_index("c")
    base = wid * _CHUNK
    pltpu.sync_copy(idx_hbm.at[pl.ds(base, _CHUNK)], idx_v)
    pltpu.async_copy(s_hbm.at[idx_v], val_v, sem).wait()
    pltpu.sync_copy(val_v, out_hbm.at[pl.ds(base, _CHUNK)])


def kernel(z, edge_index, W0, b0, W1, b1, bias):
    ei = jnp.pad(edge_index, ((0, 0), (0, _E_PAD - E)))
    ei = ei.reshape(2, _E_PAD // 128, 128)

    s, idx = pl.pallas_call(
        _tc_body,
        out_shape=[
            jax.ShapeDtypeStruct((N, N), jnp.float32),
            jax.ShapeDtypeStruct((_E_PAD // 128, 128), jnp.int32),
        ],
    )(z, ei, W0, b0.reshape(1, D), W1, b1.reshape(1, D), bias.reshape(1, 1))

    mesh = plsc.VectorSubcoreMesh(core_axis_name="c", subcore_axis_name="s")
    gather = functools.partial(
        pl.kernel,
        mesh=mesh,
        out_type=jax.ShapeDtypeStruct((_E_PAD,), jnp.float32),
        scratch_types=[
            pltpu.VMEM((_CHUNK,), jnp.int32),
            pltpu.VMEM((_CHUNK,), jnp.float32),
            pltpu.SemaphoreType.DMA,
        ],
    )(_sc_gather)

    out = gather(s.reshape(N * N), idx.reshape(_E_PAD))
    return out[:E]


# speedup vs baseline: 399.0135x; 1.3294x over previous
import functools

import jax
import jax.numpy as jnp
from jax import lax
from jax.experimental import pallas as pl
from jax.experimental.pallas import tpu as pltpu
from jax.experimental.pallas import tpu_sc as plsc

N = 1024
D = 64
E = 200000

_NW = 32
_CHUNK = 6256


def _tc_body(z_ref, ei_ref, w0_ref, b0_ref, w1_ref, b1_ref, bias_ref, s_ref,
             idx_ref):
    z = z_ref[...]
    zbar = jnp.mean(z, axis=0, keepdims=True)
    c0 = jax.nn.relu(
        jnp.dot(zbar, w0_ref[...], preferred_element_type=jnp.float32)
        + b0_ref[...])
    c1 = (jnp.dot(zbar + c0, w1_ref[...], preferred_element_type=jnp.float32)
          + b1_ref[...])
    h = z + c0 + c1
    for k in range(8):
        hk = h[128 * k:128 * (k + 1), :]
        gk = lax.dot_general(h, hk, (((1,), (1,)), ((), ())),
                             preferred_element_type=jnp.float32)
        s_ref[1024 * k:1024 * (k + 1), :] = jax.nn.sigmoid(gk + bias_ref[0, 0])
    i = ei_ref[0]
    j = ei_ref[1]
    idx_ref[...] = ((j >> 7) << 17) + (i << 7) + (j & 127)


def _sc_gather(s_hbm, idx_hbm, out_hbm, idx_v, val_v, sem):
    nc = 2
    wid = lax.axis_index("s") * nc + lax.axis_index("c")
    base = jnp.minimum(wid * _CHUNK, E - _CHUNK)
    pltpu.sync_copy(idx_hbm.at[pl.ds(base, _CHUNK)], idx_v)
    pltpu.async_copy(s_hbm.at[idx_v], val_v, sem).wait()
    pltpu.sync_copy(val_v, out_hbm.at[pl.ds(base, _CHUNK)])


def kernel(z, edge_index, W0, b0, W1, b1, bias):
    s, idx = pl.pallas_call(
        _tc_body,
        out_shape=[
            jax.ShapeDtypeStruct((8 * N, 128), jnp.float32),
            jax.ShapeDtypeStruct((E,), jnp.int32),
        ],
    )(z, edge_index, W0, b0.reshape(1, D), W1, b1.reshape(1, D),
      bias.reshape(1, 1))
    s = s.reshape(N * N)

    mesh = plsc.VectorSubcoreMesh(core_axis_name="c", subcore_axis_name="s")
    gather = functools.partial(
        pl.kernel,
        mesh=mesh,
        out_type=jax.ShapeDtypeStruct((E,), jnp.float32),
        scratch_types=[
            pltpu.VMEM((_CHUNK,), jnp.int32),
            pltpu.VMEM((_CHUNK,), jnp.float32),
            pltpu.SemaphoreType.DMA,
        ],
    )(_sc_gather)

    return gather(s, idx)
